# Initial kernel scaffold; baseline (speedup 1.0000x reference)
import jax
import jax.numpy as jnp
from jax.experimental import pallas as pl


def kernel(xyz, cost_volume, feats, flow, W1, b1, g1, be1, W2, b2, g2, be2, Wm1, bm1, Wm2, bm2, Wfc, bfc):
    raise NotImplementedError("write your pallas kernel here")


# speedup vs baseline: 26.2777x; 26.2777x over previous
import functools

import jax
import jax.numpy as jnp
from jax import lax
from jax.experimental import pallas as pl
from jax.experimental.pallas import tpu as pltpu
from jax.experimental.pallas import tpu_sc as plsc

KNB = 9
_PREC = lax.Precision.HIGHEST


def _dot(a, b):
    return lax.dot_general(a.astype(jnp.bfloat16), b.astype(jnp.bfloat16),
                           (((1,), (0,)), ((), ())),
                           preferred_element_type=jnp.float32)


def _leaky(x):
    return jnp.where(x >= 0.0, x, 0.1 * x)


def _knn_body(n_total, xt_ref, xc_ref, idx_ref):
    b = pl.program_id(0)
    xr = xt_ref[0]
    xc = xc_ref[0]
    sqr = jnp.sum(xr * xr, axis=1, keepdims=True)
    sqc = jnp.sum(xc * xc, axis=0, keepdims=True)
    cross = _dot(xr, xc)
    dist = (sqr + sqc) - 2.0 * cross
    lane = lax.broadcasted_iota(jnp.int32, dist.shape, 1)
    cols = []
    for j in range(KNB):
        m = jnp.min(dist, axis=1, keepdims=True)
        first = jnp.min(jnp.where(dist == m, lane, n_total), axis=1,
                        keepdims=True)
        cols.append(first)
        if j + 1 < KNB:
            dist = jnp.where(lane == first, 3.0e38, dist)
    idx_ref[0] = jnp.concatenate(cols, axis=1) + b * n_total


def _knn(xyzT, xyzC):
    B, N, _ = xyzT.shape
    RT = 128
    return pl.pallas_call(
        functools.partial(_knn_body, N),
        grid=(B, N // RT),
        in_specs=[
            pl.BlockSpec((1, RT, 3), lambda b, i: (b, i, 0)),
            pl.BlockSpec((1, 3, N), lambda b, i: (b, 0, 0)),
        ],
        out_specs=pl.BlockSpec((1, RT, KNB), lambda b, i: (b, i, 0)),
        out_shape=jax.ShapeDtypeStruct((B, N, KNB), jnp.int32),
    )(xyzT, xyzC)


def _proj_body(allT_ref, w1T_ref, w2xT_ref, q1_ref, px1_ref, px2_ref):
    a = allT_ref[...]
    w = w1T_ref[...]
    q1_ref[...] = _dot(a, w)
    a3 = a[:, :3]
    px1_ref[...] = _dot(a3, w[:3, :])
    px2_ref[...] = _dot(a3, w2xT_ref[...])


def _proj(allT, w1T, w2xT):
    M, Cin = allT.shape
    TP = 2048
    f32 = jnp.float32
    return pl.pallas_call(
        _proj_body,
        grid=(M // TP,),
        in_specs=[
            pl.BlockSpec((TP, Cin), lambda i: (i, 0)),
            pl.BlockSpec((Cin, 128), lambda i: (0, 0)),
            pl.BlockSpec((3, 128), lambda i: (0, 0)),
        ],
        out_specs=[pl.BlockSpec((TP, 128), lambda i: (i, 0))] * 3,
        out_shape=[jax.ShapeDtypeStruct((M, 128), f32)] * 3,
    )(allT, w1T, w2xT)


_NC, _NS = 2, 16
_NW = _NC * _NS


def _gather(qT, idxf):
    R = idxf.shape[0]
    D = qT.shape[1]
    per_w = R // _NW
    CH = 512
    n_ch = per_w // CH
    mesh = plsc.VectorSubcoreMesh(core_axis_name="c", subcore_axis_name="s")

    def body(table_hbm, idx_hbm, out_hbm, idx_v, rows_v, sem):
        wid = lax.axis_index("s") * _NC + lax.axis_index("c")
        base = wid * per_w
        for it in range(n_ch):
            off = base + it * CH
            pltpu.sync_copy(idx_hbm.at[pl.ds(off, CH)], idx_v)
            pltpu.async_copy(table_hbm.at[idx_v], rows_v, sem).wait()
            pltpu.sync_copy(rows_v, out_hbm.at[pl.ds(off, CH)])

    f = pl.kernel(
        body,
        out_type=jax.ShapeDtypeStruct((R, D), jnp.float32),
        mesh=mesh,
        scratch_types=[
            pltpu.VMEM((CH,), jnp.int32),
            pltpu.VMEM((CH, D), jnp.float32),
            pltpu.SemaphoreType.DMA,
        ],
    )
    return f(qT, idxf)


def _reduce_body(g_ref, px_ref, b_ref, mx_ref, mn_ref, sp_ref, ssp_ref):
    g0 = g_ref[0]
    mx = g0
    mn = g0
    s = g0
    ss = g0 * g0
    for j in range(1, KNB):
        gj = g_ref[j]
        mx = jnp.maximum(mx, gj)
        mn = jnp.minimum(mn, gj)
        s = s + gj
        ss = ss + gj * gj
    mx_ref[...] = mx
    mn_ref[...] = mn
    pb = px_ref[...] - b_ref[...]
    kf = float(KNB)
    srow = s - kf * pb
    ssrow = ss - 2.0 * pb * s + kf * (pb * pb)
    sp_ref[...] = jnp.sum(srow, axis=0, keepdims=True)[None]
    ssp_ref[...] = jnp.sum(ssrow, axis=0, keepdims=True)[None]


def _kreduce(G3, pxT, bvec):
    _, M, D = G3.shape
    TP = 512
    NT = M // TP
    f32 = jnp.float32
    return pl.pallas_call(
        _reduce_body,
        grid=(NT,),
        in_specs=[
            pl.BlockSpec((KNB, TP, D), lambda i: (0, i, 0)),
            pl.BlockSpec((TP, D), lambda i: (i, 0)),
            pl.BlockSpec((1, D), lambda i: (0, 0)),
        ],
        out_specs=[
            pl.BlockSpec((TP, D), lambda i: (i, 0)),
            pl.BlockSpec((TP, D), lambda i: (i, 0)),
            pl.BlockSpec((1, 1, D), lambda i: (i, 0, 0)),
            pl.BlockSpec((1, 1, D), lambda i: (i, 0, 0)),
        ],
        out_shape=[
            jax.ShapeDtypeStruct((M, D), f32),
            jax.ShapeDtypeStruct((M, D), f32),
            jax.ShapeDtypeStruct((NT, 1, D), f32),
            jax.ShapeDtypeStruct((NT, 1, D), f32),
        ],
    )(G3, pxT, bvec)


def _bn_sel(count, sp, ssp, mx, mn, px, b, g, be):
    tot = jnp.sum(sp, axis=0)
    tot2 = jnp.sum(ssp, axis=0)
    mean = tot * (1.0 / count)
    var = tot2 * (1.0 / count) - mean * mean
    scale = g * lax.rsqrt(var + 1e-5)
    xsel = jnp.where(g >= 0.0, mx, mn) - (px - b)
    return _leaky((xsel - mean) * scale + be)


def _fin1_body(count, sp_ref, ssp_ref, mx_ref, mn_ref, px1_ref, b_ref, g_ref,
               be_ref, px2_ref, w2pT_ref, q2_ref):
    y = _bn_sel(count, sp_ref[...], ssp_ref[...], mx_ref[...], mn_ref[...],
                px1_ref[...], b_ref[...], g_ref[...], be_ref[...])
    q2_ref[...] = _dot(y, w2pT_ref[...]) + px2_ref[...]


def _fin1(count, sp, ssp, mx, mn, px1T, b1, g1, be1, px2T, w2pT):
    M, D = mx.shape
    TP = 2048
    NT1 = sp.shape[0]
    return pl.pallas_call(
        functools.partial(_fin1_body, count),
        grid=(M // TP,),
        in_specs=[
            pl.BlockSpec((NT1, 1, D), lambda i: (0, 0, 0)),
            pl.BlockSpec((NT1, 1, D), lambda i: (0, 0, 0)),
            pl.BlockSpec((TP, D), lambda i: (i, 0)),
            pl.BlockSpec((TP, D), lambda i: (i, 0)),
            pl.BlockSpec((TP, D), lambda i: (i, 0)),
            pl.BlockSpec((1, D), lambda i: (0, 0)),
            pl.BlockSpec((1, D), lambda i: (0, 0)),
            pl.BlockSpec((1, D), lambda i: (0, 0)),
            pl.BlockSpec((TP, D), lambda i: (i, 0)),
            pl.BlockSpec((D, D), lambda i: (0, 0)),
        ],
        out_specs=pl.BlockSpec((TP, D), lambda i: (i, 0)),
        out_shape=jax.ShapeDtypeStruct((M, D), jnp.float32),
    )(sp, ssp, mx, mn, px1T, b1, g1, be1, px2T, w2pT)


def _fin2_body(count, sp_ref, ssp_ref, mx_ref, mn_ref, px2_ref, b_ref, g_ref,
               be_ref, wm1T_ref, bm1_ref, wm2T_ref, bm2_ref, wfcT_ref,
               bfc_ref, np_ref, rf_ref):
    y = _bn_sel(count, sp_ref[...], ssp_ref[...], mx_ref[...], mn_ref[...],
                px2_ref[...], b_ref[...], g_ref[...], be_ref[...])
    h1 = _leaky(_dot(y, wm1T_ref[...]) + bm1_ref[...])
    h2 = _leaky(_dot(h1, wm2T_ref[...]) + bm2_ref[...])
    rf = _dot(h2, wfcT_ref[...]) + bfc_ref[...]
    np_ref[...] = h2
    rf_ref[...] = jnp.clip(rf, -20.0, 20.0)


def _fin2(count, sp, ssp, mx, mn, px2T, b2, g2, be2, wm1T, bm1, wm2T, bm2,
          wfcT, bfc):
    M, D = mx.shape
    TP = 2048
    NT1 = sp.shape[0]
    Dm = wm2T.shape[1]
    Do = wfcT.shape[1]
    return pl.pallas_call(
        functools.partial(_fin2_body, count),
        grid=(M // TP,),
        in_specs=[
            pl.BlockSpec((NT1, 1, D), lambda i: (0, 0, 0)),
            pl.BlockSpec((NT1, 1, D), lambda i: (0, 0, 0)),
            pl.BlockSpec((TP, D), lambda i: (i, 0)),
            pl.BlockSpec((TP, D), lambda i: (i, 0)),
            pl.BlockSpec((TP, D), lambda i: (i, 0)),
            pl.BlockSpec((1, D), lambda i: (0, 0)),
            pl.BlockSpec((1, D), lambda i: (0, 0)),
            pl.BlockSpec((1, D), lambda i: (0, 0)),
            pl.BlockSpec((D, D), lambda i: (0, 0)),
            pl.BlockSpec((1, D), lambda i: (0, 0)),
            pl.BlockSpec((D, Dm), lambda i: (0, 0)),
            pl.BlockSpec((1, Dm), lambda i: (0, 0)),
            pl.BlockSpec((Dm, Do), lambda i: (0, 0)),
            pl.BlockSpec((1, Do), lambda i: (0, 0)),
        ],
        out_specs=[
            pl.BlockSpec((TP, Dm), lambda i: (i, 0)),
            pl.BlockSpec((TP, Do), lambda i: (i, 0)),
        ],
        out_shape=[
            jax.ShapeDtypeStruct((M, Dm), jnp.float32),
            jax.ShapeDtypeStruct((M, Do), jnp.float32),
        ],
    )(sp, ssp, mx, mn, px2T, b2, g2, be2, wm1T, bm1, wm2T, bm2, wfcT, bfc)


def kernel(xyz, cost_volume, feats, flow, W1, b1, g1, be1, W2, b2, g2, be2,
           Wm1, bm1, Wm2, bm2, Wfc, bfc):
    B, _, N = xyz.shape
    M = B * N
    count = float(M * KNB)

    xyzT = jnp.transpose(xyz, (0, 2, 1))
    ptsT = jnp.transpose(jnp.concatenate([feats, cost_volume, flow], axis=1),
                         (0, 2, 1))
    allT = jnp.concatenate([xyzT, ptsT], axis=-1).reshape(M, -1)

    idx = _knn(xyzT, xyz)
    idxf = jnp.transpose(idx.reshape(M, KNB), (1, 0)).reshape(-1)

    q1T, px1T, px2T = _proj(allT, jnp.transpose(W1, (1, 0)),
                            jnp.transpose(W2[:, :3], (1, 0)))

    r1 = lambda v: v.reshape(1, -1)
    G1 = _gather(q1T, idxf).reshape(KNB, M, 128)
    mx1, mn1, sp1, ssp1 = _kreduce(G1, px1T, r1(b1))
    q2T = _fin1(count, sp1, ssp1, mx1, mn1, px1T, r1(b1), r1(g1), r1(be1),
                px2T, jnp.transpose(W2[:, 3:], (1, 0)))

    G2 = _gather(q2T, idxf).reshape(KNB, M, 128)
    mx2, mn2, sp2, ssp2 = _kreduce(G2, px2T, r1(b2))
    npT, rfT = _fin2(count, sp2, ssp2, mx2, mn2, px2T, r1(b2), r1(g2),
                     r1(be2), jnp.transpose(Wm1, (1, 0)), r1(bm1),
                     jnp.transpose(Wm2, (1, 0)), r1(bm2),
                     jnp.transpose(Wfc, (1, 0)), r1(bfc))

    new_points = jnp.transpose(npT.reshape(B, N, -1), (0, 2, 1))
    re_flow = jnp.transpose(rfT.reshape(B, N, -1), (0, 2, 1))
    return (new_points, re_flow)


# speedup vs baseline: 36.4911x; 1.3887x over previous
import functools

import jax
import jax.numpy as jnp
from jax import lax
from jax.experimental import pallas as pl
from jax.experimental.pallas import tpu as pltpu
from jax.experimental.pallas import tpu_sc as plsc

KNB = 9
_PREC = lax.Precision.HIGHEST


def _dot(a, b):
    return lax.dot_general(a.astype(jnp.bfloat16), b.astype(jnp.bfloat16),
                           (((1,), (0,)), ((), ())),
                           preferred_element_type=jnp.float32)


def _leaky(x):
    return jnp.where(x >= 0.0, x, 0.1 * x)


def _knn_body(n_total, rt, xr_ref, xam_ref, sq_ref, idx_ref):
    b = pl.program_id(0)
    xr = xr_ref[0]
    xam = xam_ref[0]
    sqm = sq_ref[0]
    sqr = jnp.sum(xr * xr, axis=0, keepdims=True)
    cross = _dot(xam, xr)
    dist = (sqm + sqr) - 2.0 * cross
    ng = n_total // 128
    cand = lax.broadcasted_iota(jnp.int32, (n_total, rt), 0)
    c7 = jnp.bitwise_and(cand, 127)
    bits = lax.bitcast_convert_type(jnp.maximum(dist, 1.0e-35), jnp.int32)
    keys = lax.bitcast_convert_type(
        jnp.bitwise_or(jnp.bitwise_and(bits, jnp.int32(-128)), c7),
        jnp.float32)
    g_iota = lax.broadcasted_iota(jnp.int32, (ng, rt), 0)
    gmin = jnp.min(keys.reshape(ng, 128, rt), axis=1)
    rows = []
    for j in range(KNB):
        m = jnp.min(gmin, axis=0, keepdims=True)
        gj = jnp.min(jnp.where(gmin == m, g_iota, ng), axis=0, keepdims=True)
        cj = jnp.bitwise_and(lax.bitcast_convert_type(m, jnp.int32), 127)
        sel = gj * 128 + cj
        rows.append(sel)
        if j + 1 < KNB:
            keys = jnp.where(cand == sel, 3.0e38, keys)
            gmin = jnp.min(keys.reshape(ng, 128, rt), axis=1)
    idx_ref[0] = jnp.concatenate(rows, axis=0) + b * n_total


def _knn(xyzC, xyzT, sqT):
    B, _, N = xyzC.shape
    RT = 128
    return pl.pallas_call(
        functools.partial(_knn_body, N, RT),
        grid=(B, N // RT),
        in_specs=[
            pl.BlockSpec((1, 3, RT), lambda b, i: (b, 0, i)),
            pl.BlockSpec((1, N, 3), lambda b, i: (b, 0, 0)),
            pl.BlockSpec((1, N, 1), lambda b, i: (b, 0, 0)),
        ],
        out_specs=pl.BlockSpec((1, KNB, RT), lambda b, i: (b, 0, i)),
        out_shape=jax.ShapeDtypeStruct((B, KNB, N), jnp.int32),
    )(xyzC, xyzT, sqT)


def _proj_body(allT_ref, w1T_ref, w2xT_ref, q1_ref, px1_ref, px2_ref, sq_ref):
    a = allT_ref[...]
    w = w1T_ref[...]
    q1_ref[...] = _dot(a, w)
    a3 = a[:, :3]
    px1_ref[...] = _dot(a3, w[:3, :])
    px2_ref[...] = _dot(a3, w2xT_ref[...])
    sq_ref[...] = jnp.sum(a3 * a3, axis=1, keepdims=True)


def _proj(allT, w1T, w2xT):
    M, Cin = allT.shape
    TP = 2048
    f32 = jnp.float32
    return pl.pallas_call(
        _proj_body,
        grid=(M // TP,),
        in_specs=[
            pl.BlockSpec((TP, Cin), lambda i: (i, 0)),
            pl.BlockSpec((Cin, 128), lambda i: (0, 0)),
            pl.BlockSpec((3, 128), lambda i: (0, 0)),
        ],
        out_specs=[pl.BlockSpec((TP, 128), lambda i: (i, 0))] * 3
        + [pl.BlockSpec((TP, 1), lambda i: (i, 0))],
        out_shape=[jax.ShapeDtypeStruct((M, 128), f32)] * 3
        + [jax.ShapeDtypeStruct((M, 1), f32)],
    )(allT, w1T, w2xT)


_NC, _NS = 2, 16
_NW = _NC * _NS


def _gather(qT, idxf):
    R = idxf.shape[0]
    D = qT.shape[1]
    per_w = R // _NW
    CH = 512
    n_ch = per_w // CH
    mesh = plsc.VectorSubcoreMesh(core_axis_name="c", subcore_axis_name="s")

    def body(table_hbm, idx_hbm, out_hbm, idx_v, rows_v, sem):
        wid = lax.axis_index("s") * _NC + lax.axis_index("c")
        base = wid * per_w
        for it in range(n_ch):
            off = base + it * CH
            pltpu.sync_copy(idx_hbm.at[pl.ds(off, CH)], idx_v)
            pltpu.async_copy(table_hbm.at[idx_v], rows_v, sem).wait()
            pltpu.sync_copy(rows_v, out_hbm.at[pl.ds(off, CH)])

    f = pl.kernel(
        body,
        out_type=jax.ShapeDtypeStruct((R, D), jnp.float32),
        mesh=mesh,
        scratch_types=[
            pltpu.VMEM((CH,), jnp.int32),
            pltpu.VMEM((CH, D), jnp.float32),
            pltpu.SemaphoreType.DMA,
        ],
    )
    return f(qT, idxf)


def _reduce_body(g_ref, px_ref, b_ref, mx_ref, mn_ref, sp_ref, ssp_ref):
    g0 = g_ref[0]
    mx = g0
    mn = g0
    s = g0
    ss = g0 * g0
    for j in range(1, KNB):
        gj = g_ref[j]
        mx = jnp.maximum(mx, gj)
        mn = jnp.minimum(mn, gj)
        s = s + gj
        ss = ss + gj * gj
    mx_ref[...] = mx
    mn_ref[...] = mn
    pb = px_ref[...] - b_ref[...]
    kf = float(KNB)
    srow = s - kf * pb
    ssrow = ss - 2.0 * pb * s + kf * (pb * pb)
    sp_ref[...] = jnp.sum(srow, axis=0, keepdims=True)[None]
    ssp_ref[...] = jnp.sum(ssrow, axis=0, keepdims=True)[None]


def _kreduce(G3, pxT, bvec):
    _, M, D = G3.shape
    TP = 512
    NT = M // TP
    f32 = jnp.float32
    return pl.pallas_call(
        _reduce_body,
        grid=(NT,),
        in_specs=[
            pl.BlockSpec((KNB, TP, D), lambda i: (0, i, 0)),
            pl.BlockSpec((TP, D), lambda i: (i, 0)),
            pl.BlockSpec((1, D), lambda i: (0, 0)),
        ],
        out_specs=[
            pl.BlockSpec((TP, D), lambda i: (i, 0)),
            pl.BlockSpec((TP, D), lambda i: (i, 0)),
            pl.BlockSpec((1, 1, D), lambda i: (i, 0, 0)),
            pl.BlockSpec((1, 1, D), lambda i: (i, 0, 0)),
        ],
        out_shape=[
            jax.ShapeDtypeStruct((M, D), f32),
            jax.ShapeDtypeStruct((M, D), f32),
            jax.ShapeDtypeStruct((NT, 1, D), f32),
            jax.ShapeDtypeStruct((NT, 1, D), f32),
        ],
    )(G3, pxT, bvec)


def _bn_sel(count, sp, ssp, mx, mn, px, b, g, be):
    tot = jnp.sum(sp, axis=0)
    tot2 = jnp.sum(ssp, axis=0)
    mean = tot * (1.0 / count)
    var = tot2 * (1.0 / count) - mean * mean
    scale = g * lax.rsqrt(var + 1e-5)
    xsel = jnp.where(g >= 0.0, mx, mn) - (px - b)
    return _leaky((xsel - mean) * scale + be)


def _fin1_body(count, sp_ref, ssp_ref, mx_ref, mn_ref, px1_ref, b_ref, g_ref,
               be_ref, px2_ref, w2pT_ref, q2_ref):
    y = _bn_sel(count, sp_ref[...], ssp_ref[...], mx_ref[...], mn_ref[...],
                px1_ref[...], b_ref[...], g_ref[...], be_ref[...])
    q2_ref[...] = _dot(y, w2pT_ref[...]) + px2_ref[...]


def _fin1(count, sp, ssp, mx, mn, px1T, b1, g1, be1, px2T, w2pT):
    M, D = mx.shape
    TP = 2048
    NT1 = sp.shape[0]
    return pl.pallas_call(
        functools.partial(_fin1_body, count),
        grid=(M // TP,),
        in_specs=[
            pl.BlockSpec((NT1, 1, D), lambda i: (0, 0, 0)),
            pl.BlockSpec((NT1, 1, D), lambda i: (0, 0, 0)),
            pl.BlockSpec((TP, D), lambda i: (i, 0)),
            pl.BlockSpec((TP, D), lambda i: (i, 0)),
            pl.BlockSpec((TP, D), lambda i: (i, 0)),
            pl.BlockSpec((1, D), lambda i: (0, 0)),
            pl.BlockSpec((1, D), lambda i: (0, 0)),
            pl.BlockSpec((1, D), lambda i: (0, 0)),
            pl.BlockSpec((TP, D), lambda i: (i, 0)),
            pl.BlockSpec((D, D), lambda i: (0, 0)),
        ],
        out_specs=pl.BlockSpec((TP, D), lambda i: (i, 0)),
        out_shape=jax.ShapeDtypeStruct((M, D), jnp.float32),
    )(sp, ssp, mx, mn, px1T, b1, g1, be1, px2T, w2pT)


def _fin2_body(count, sp_ref, ssp_ref, mx_ref, mn_ref, px2_ref, b_ref, g_ref,
               be_ref, wm1T_ref, bm1_ref, wm2T_ref, bm2_ref, wfcT_ref,
               bfc_ref, np_ref, rf_ref):
    y = _bn_sel(count, sp_ref[...], ssp_ref[...], mx_ref[...], mn_ref[...],
                px2_ref[...], b_ref[...], g_ref[...], be_ref[...])
    h1 = _leaky(_dot(y, wm1T_ref[...]) + bm1_ref[...])
    h2 = _leaky(_dot(h1, wm2T_ref[...]) + bm2_ref[...])
    rf = _dot(h2, wfcT_ref[...]) + bfc_ref[...]
    np_ref[...] = h2
    rf_ref[...] = jnp.clip(rf, -20.0, 20.0)


def _fin2(count, sp, ssp, mx, mn, px2T, b2, g2, be2, wm1T, bm1, wm2T, bm2,
          wfcT, bfc):
    M, D = mx.shape
    TP = 2048
    NT1 = sp.shape[0]
    Dm = wm2T.shape[1]
    Do = wfcT.shape[1]
    return pl.pallas_call(
        functools.partial(_fin2_body, count),
        grid=(M // TP,),
        in_specs=[
            pl.BlockSpec((NT1, 1, D), lambda i: (0, 0, 0)),
            pl.BlockSpec((NT1, 1, D), lambda i: (0, 0, 0)),
            pl.BlockSpec((TP, D), lambda i: (i, 0)),
            pl.BlockSpec((TP, D), lambda i: (i, 0)),
            pl.BlockSpec((TP, D), lambda i: (i, 0)),
            pl.BlockSpec((1, D), lambda i: (0, 0)),
            pl.BlockSpec((1, D), lambda i: (0, 0)),
            pl.BlockSpec((1, D), lambda i: (0, 0)),
            pl.BlockSpec((D, D), lambda i: (0, 0)),
            pl.BlockSpec((1, D), lambda i: (0, 0)),
            pl.BlockSpec((D, Dm), lambda i: (0, 0)),
            pl.BlockSpec((1, Dm), lambda i: (0, 0)),
            pl.BlockSpec((Dm, Do), lambda i: (0, 0)),
            pl.BlockSpec((1, Do), lambda i: (0, 0)),
        ],
        out_specs=[
            pl.BlockSpec((TP, Dm), lambda i: (i, 0)),
            pl.BlockSpec((TP, Do), lambda i: (i, 0)),
        ],
        out_shape=[
            jax.ShapeDtypeStruct((M, Dm), jnp.float32),
            jax.ShapeDtypeStruct((M, Do), jnp.float32),
        ],
    )(sp, ssp, mx, mn, px2T, b2, g2, be2, wm1T, bm1, wm2T, bm2, wfcT, bfc)


def kernel(xyz, cost_volume, feats, flow, W1, b1, g1, be1, W2, b2, g2, be2,
           Wm1, bm1, Wm2, bm2, Wfc, bfc):
    B, _, N = xyz.shape
    M = B * N
    count = float(M * KNB)

    xyzT = jnp.transpose(xyz, (0, 2, 1))
    ptsT = jnp.transpose(jnp.concatenate([feats, cost_volume, flow], axis=1),
                         (0, 2, 1))
    allT = jnp.concatenate([xyzT, ptsT], axis=-1).reshape(M, -1)

    q1T, px1T, px2T, sqF = _proj(allT, jnp.transpose(W1, (1, 0)),
                                 jnp.transpose(W2[:, :3], (1, 0)))

    idx = _knn(xyz, xyzT, sqF.reshape(B, N, 1))
    idxf = jnp.transpose(idx, (1, 0, 2)).reshape(-1)

    r1 = lambda v: v.reshape(1, -1)
    G1 = _gather(q1T, idxf).reshape(KNB, M, 128)
    mx1, mn1, sp1, ssp1 = _kreduce(G1, px1T, r1(b1))
    q2T = _fin1(count, sp1, ssp1, mx1, mn1, px1T, r1(b1), r1(g1), r1(be1),
                px2T, jnp.transpose(W2[:, 3:], (1, 0)))

    G2 = _gather(q2T, idxf).reshape(KNB, M, 128)
    mx2, mn2, sp2, ssp2 = _kreduce(G2, px2T, r1(b2))
    npT, rfT = _fin2(count, sp2, ssp2, mx2, mn2, px2T, r1(b2), r1(g2),
                     r1(be2), jnp.transpose(Wm1, (1, 0)), r1(bm1),
                     jnp.transpose(Wm2, (1, 0)), r1(bm2),
                     jnp.transpose(Wfc, (1, 0)), r1(bfc))

    new_points = jnp.transpose(npT.reshape(B, N, -1), (0, 2, 1))
    re_flow = jnp.transpose(rfT.reshape(B, N, -1), (0, 2, 1))
    return (new_points, re_flow)


# speedup vs baseline: 37.6472x; 1.0317x over previous
import functools

import jax
import jax.numpy as jnp
from jax import lax
from jax.experimental import pallas as pl
from jax.experimental.pallas import tpu as pltpu
from jax.experimental.pallas import tpu_sc as plsc

KNB = 9
_PREC = lax.Precision.HIGHEST


def _dot(a, b):
    return lax.dot_general(a.astype(jnp.bfloat16), b.astype(jnp.bfloat16),
                           (((1,), (0,)), ((), ())),
                           preferred_element_type=jnp.float32)


def _leaky(x):
    return jnp.where(x >= 0.0, x, 0.1 * x)


def _knn_body(n_total, rt, xr_ref, xam_ref, sq_ref, idx_ref):
    b = pl.program_id(0)
    xr = xr_ref[0]
    xam = xam_ref[0]
    sqm = sq_ref[0]
    sqr = jnp.sum(xr * xr, axis=0, keepdims=True)
    cross = _dot(xam, xr)
    dist = (sqm + sqr) - 2.0 * cross
    ng = n_total // 128
    cand = lax.broadcasted_iota(jnp.int32, (n_total, rt), 0)
    c7 = jnp.bitwise_and(cand, 127)
    bits = lax.bitcast_convert_type(jnp.maximum(dist, 1.0e-35), jnp.int32)
    keys = lax.bitcast_convert_type(
        jnp.bitwise_or(jnp.bitwise_and(bits, jnp.int32(-128)), c7),
        jnp.float32)
    g_iota = lax.broadcasted_iota(jnp.int32, (ng, rt), 0)
    gmin = jnp.min(keys.reshape(ng, 128, rt), axis=1)
    rows = []
    for j in range(KNB):
        m = jnp.min(gmin, axis=0, keepdims=True)
        gj = jnp.min(jnp.where(gmin == m, g_iota, ng), axis=0, keepdims=True)
        cj = jnp.bitwise_and(lax.bitcast_convert_type(m, jnp.int32), 127)
        sel = gj * 128 + cj
        rows.append(sel)
        if j + 1 < KNB:
            keys = jnp.where(cand == sel, 3.0e38, keys)
            gmin = jnp.min(keys.reshape(ng, 128, rt), axis=1)
    idx_ref[0] = jnp.concatenate(rows, axis=0) + b * n_total


def _knn(xyzC, xyzT, sqT):
    B, _, N = xyzC.shape
    RT = 128
    return pl.pallas_call(
        functools.partial(_knn_body, N, RT),
        grid=(B, N // RT),
        in_specs=[
            pl.BlockSpec((1, 3, RT), lambda b, i: (b, 0, i)),
            pl.BlockSpec((1, N, 3), lambda b, i: (b, 0, 0)),
            pl.BlockSpec((1, N, 1), lambda b, i: (b, 0, 0)),
        ],
        out_specs=pl.BlockSpec((1, KNB, RT), lambda b, i: (b, 0, i)),
        out_shape=jax.ShapeDtypeStruct((B, KNB, N), jnp.int32),
    )(xyzC, xyzT, sqT)


def _proj_body(xyz_ref, fe_ref, cv_ref, fl_ref, w1_ref, w2x_ref,
               q1_ref, px1_ref, px2_ref, sq_ref):
    xc = xyz_ref[0]
    w1 = w1_ref[...]
    px1 = _dot(w1[:, :3], xc)
    q1 = (px1 + _dot(w1[:, 3:131], fe_ref[0])
          + _dot(w1[:, 131:195], cv_ref[0])
          + _dot(w1[:, 195:], fl_ref[0]))
    px2 = _dot(w2x_ref[...], xc)
    q1_ref[0] = jnp.transpose(q1, (1, 0))
    px1_ref[0] = jnp.transpose(px1, (1, 0))
    px2_ref[0] = jnp.transpose(px2, (1, 0))
    sq_ref[0] = jnp.transpose(jnp.sum(xc * xc, axis=0, keepdims=True), (1, 0))


def _proj(xyz, feats, cost_volume, flow, W1, W2x):
    B, _, N = xyz.shape
    TN = 2048
    f32 = jnp.float32
    return pl.pallas_call(
        _proj_body,
        grid=(B, N // TN),
        in_specs=[
            pl.BlockSpec((1, 3, TN), lambda b, i: (b, 0, i)),
            pl.BlockSpec((1, 128, TN), lambda b, i: (b, 0, i)),
            pl.BlockSpec((1, 64, TN), lambda b, i: (b, 0, i)),
            pl.BlockSpec((1, 3, TN), lambda b, i: (b, 0, i)),
            pl.BlockSpec((128, 198), lambda b, i: (0, 0)),
            pl.BlockSpec((128, 3), lambda b, i: (0, 0)),
        ],
        out_specs=[pl.BlockSpec((1, TN, 128), lambda b, i: (b, i, 0))] * 3
        + [pl.BlockSpec((1, TN, 1), lambda b, i: (b, i, 0))],
        out_shape=[jax.ShapeDtypeStruct((B, N, 128), f32)] * 3
        + [jax.ShapeDtypeStruct((B, N, 1), f32)],
    )(xyz, feats, cost_volume, flow, W1, W2x)


_NC, _NS = 2, 16
_NW = _NC * _NS


def _gather(qT, idxf):
    R = idxf.shape[0]
    D = qT.shape[1]
    per_w = R // _NW
    CH = 512
    n_ch = per_w // CH
    mesh = plsc.VectorSubcoreMesh(core_axis_name="c", subcore_axis_name="s")

    def body(table_hbm, idx_hbm, out_hbm, idx_v, rows_v, sem):
        wid = lax.axis_index("s") * _NC + lax.axis_index("c")
        base = wid * per_w
        for it in range(n_ch):
            off = base + it * CH
            pltpu.sync_copy(idx_hbm.at[pl.ds(off, CH)], idx_v)
            pltpu.async_copy(table_hbm.at[idx_v], rows_v, sem).wait()
            pltpu.sync_copy(rows_v, out_hbm.at[pl.ds(off, CH)])

    f = pl.kernel(
        body,
        out_type=jax.ShapeDtypeStruct((R, D), jnp.float32),
        mesh=mesh,
        scratch_types=[
            pltpu.VMEM((CH,), jnp.int32),
            pltpu.VMEM((CH, D), jnp.float32),
            pltpu.SemaphoreType.DMA,
        ],
    )
    return f(qT, idxf)


def _reduce_body(g_ref, px_ref, b_ref, mx_ref, mn_ref, sp_ref, ssp_ref):
    g0 = g_ref[0]
    mx = g0
    mn = g0
    s = g0
    ss = g0 * g0
    for j in range(1, KNB):
        gj = g_ref[j]
        mx = jnp.maximum(mx, gj)
        mn = jnp.minimum(mn, gj)
        s = s + gj
        ss = ss + gj * gj
    mx_ref[...] = mx
    mn_ref[...] = mn
    pb = px_ref[...] - b_ref[...]
    kf = float(KNB)
    srow = s - kf * pb
    ssrow = ss - 2.0 * pb * s + kf * (pb * pb)
    sp_ref[...] = jnp.sum(srow, axis=0, keepdims=True)[None]
    ssp_ref[...] = jnp.sum(ssrow, axis=0, keepdims=True)[None]


def _kreduce(G3, pxT, bvec):
    _, M, D = G3.shape
    TP = 512
    NT = M // TP
    f32 = jnp.float32
    return pl.pallas_call(
        _reduce_body,
        grid=(NT,),
        in_specs=[
            pl.BlockSpec((KNB, TP, D), lambda i: (0, i, 0)),
            pl.BlockSpec((TP, D), lambda i: (i, 0)),
            pl.BlockSpec((1, D), lambda i: (0, 0)),
        ],
        out_specs=[
            pl.BlockSpec((TP, D), lambda i: (i, 0)),
            pl.BlockSpec((TP, D), lambda i: (i, 0)),
            pl.BlockSpec((1, 1, D), lambda i: (i, 0, 0)),
            pl.BlockSpec((1, 1, D), lambda i: (i, 0, 0)),
        ],
        out_shape=[
            jax.ShapeDtypeStruct((M, D), f32),
            jax.ShapeDtypeStruct((M, D), f32),
            jax.ShapeDtypeStruct((NT, 1, D), f32),
            jax.ShapeDtypeStruct((NT, 1, D), f32),
        ],
    )(G3, pxT, bvec)


def _bn_sel(count, sp, ssp, mx, mn, px, b, g, be):
    tot = jnp.sum(sp, axis=0)
    tot2 = jnp.sum(ssp, axis=0)
    mean = tot * (1.0 / count)
    var = tot2 * (1.0 / count) - mean * mean
    scale = g * lax.rsqrt(var + 1e-5)
    xsel = jnp.where(g >= 0.0, mx, mn) - (px - b)
    return _leaky((xsel - mean) * scale + be)


def _fin1_body(count, sp_ref, ssp_ref, mx_ref, mn_ref, px1_ref, b_ref, g_ref,
               be_ref, px2_ref, w2pT_ref, q2_ref):
    y = _bn_sel(count, sp_ref[...], ssp_ref[...], mx_ref[...], mn_ref[...],
                px1_ref[...], b_ref[...], g_ref[...], be_ref[...])
    q2_ref[...] = _dot(y, w2pT_ref[...]) + px2_ref[...]


def _fin1(count, sp, ssp, mx, mn, px1T, b1, g1, be1, px2T, w2pT):
    M, D = mx.shape
    TP = 2048
    NT1 = sp.shape[0]
    return pl.pallas_call(
        functools.partial(_fin1_body, count),
        grid=(M // TP,),
        in_specs=[
            pl.BlockSpec((NT1, 1, D), lambda i: (0, 0, 0)),
            pl.BlockSpec((NT1, 1, D), lambda i: (0, 0, 0)),
            pl.BlockSpec((TP, D), lambda i: (i, 0)),
            pl.BlockSpec((TP, D), lambda i: (i, 0)),
            pl.BlockSpec((TP, D), lambda i: (i, 0)),
            pl.BlockSpec((1, D), lambda i: (0, 0)),
            pl.BlockSpec((1, D), lambda i: (0, 0)),
            pl.BlockSpec((1, D), lambda i: (0, 0)),
            pl.BlockSpec((TP, D), lambda i: (i, 0)),
            pl.BlockSpec((D, D), lambda i: (0, 0)),
        ],
        out_specs=pl.BlockSpec((TP, D), lambda i: (i, 0)),
        out_shape=jax.ShapeDtypeStruct((M, D), jnp.float32),
    )(sp, ssp, mx, mn, px1T, b1, g1, be1, px2T, w2pT)


def _fin2_body(count, sp_ref, ssp_ref, mx_ref, mn_ref, px2_ref, b_ref, g_ref,
               be_ref, wm1T_ref, bm1_ref, wm2T_ref, bm2_ref, wfcT_ref,
               bfc_ref, np_ref, rf_ref):
    y = _bn_sel(count, sp_ref[...], ssp_ref[...], mx_ref[...], mn_ref[...],
                px2_ref[...], b_ref[...], g_ref[...], be_ref[...])
    h1 = _leaky(_dot(y, wm1T_ref[...]) + bm1_ref[...])
    h2 = _leaky(_dot(h1, wm2T_ref[...]) + bm2_ref[...])
    rf = _dot(h2, wfcT_ref[...]) + bfc_ref[...]
    np_ref[...] = h2
    rf_ref[...] = jnp.clip(rf, -20.0, 20.0)


def _fin2(count, sp, ssp, mx, mn, px2T, b2, g2, be2, wm1T, bm1, wm2T, bm2,
          wfcT, bfc):
    M, D = mx.shape
    TP = 2048
    NT1 = sp.shape[0]
    Dm = wm2T.shape[1]
    Do = wfcT.shape[1]
    return pl.pallas_call(
        functools.partial(_fin2_body, count),
        grid=(M // TP,),
        in_specs=[
            pl.BlockSpec((NT1, 1, D), lambda i: (0, 0, 0)),
            pl.BlockSpec((NT1, 1, D), lambda i: (0, 0, 0)),
            pl.BlockSpec((TP, D), lambda i: (i, 0)),
            pl.BlockSpec((TP, D), lambda i: (i, 0)),
            pl.BlockSpec((TP, D), lambda i: (i, 0)),
            pl.BlockSpec((1, D), lambda i: (0, 0)),
            pl.BlockSpec((1, D), lambda i: (0, 0)),
            pl.BlockSpec((1, D), lambda i: (0, 0)),
            pl.BlockSpec((D, D), lambda i: (0, 0)),
            pl.BlockSpec((1, D), lambda i: (0, 0)),
            pl.BlockSpec((D, Dm), lambda i: (0, 0)),
            pl.BlockSpec((1, Dm), lambda i: (0, 0)),
            pl.BlockSpec((Dm, Do), lambda i: (0, 0)),
            pl.BlockSpec((1, Do), lambda i: (0, 0)),
        ],
        out_specs=[
            pl.BlockSpec((TP, Dm), lambda i: (i, 0)),
            pl.BlockSpec((TP, Do), lambda i: (i, 0)),
        ],
        out_shape=[
            jax.ShapeDtypeStruct((M, Dm), jnp.float32),
            jax.ShapeDtypeStruct((M, Do), jnp.float32),
        ],
    )(sp, ssp, mx, mn, px2T, b2, g2, be2, wm1T, bm1, wm2T, bm2, wfcT, bfc)


def kernel(xyz, cost_volume, feats, flow, W1, b1, g1, be1, W2, b2, g2, be2,
           Wm1, bm1, Wm2, bm2, Wfc, bfc):
    B, _, N = xyz.shape
    M = B * N
    count = float(M * KNB)

    xyzT = jnp.transpose(xyz, (0, 2, 1))

    q1, px1, px2, sq = _proj(xyz, feats, cost_volume, flow, W1, W2[:, :3])
    q1T = q1.reshape(M, 128)
    px1T = px1.reshape(M, 128)
    px2T = px2.reshape(M, 128)

    idx = _knn(xyz, xyzT, sq)
    idxf = jnp.transpose(idx, (1, 0, 2)).reshape(-1)

    r1 = lambda v: v.reshape(1, -1)
    G1 = _gather(q1T, idxf).reshape(KNB, M, 128)
    mx1, mn1, sp1, ssp1 = _kreduce(G1, px1T, r1(b1))
    q2T = _fin1(count, sp1, ssp1, mx1, mn1, px1T, r1(b1), r1(g1), r1(be1),
                px2T, jnp.transpose(W2[:, 3:], (1, 0)))

    G2 = _gather(q2T, idxf).reshape(KNB, M, 128)
    mx2, mn2, sp2, ssp2 = _kreduce(G2, px2T, r1(b2))
    npT, rfT = _fin2(count, sp2, ssp2, mx2, mn2, px2T, r1(b2), r1(g2),
                     r1(be2), jnp.transpose(Wm1, (1, 0)), r1(bm1),
                     jnp.transpose(Wm2, (1, 0)), r1(bm2),
                     jnp.transpose(Wfc, (1, 0)), r1(bfc))

    new_points = jnp.transpose(npT.reshape(B, N, -1), (0, 2, 1))
    re_flow = jnp.transpose(rfT.reshape(B, N, -1), (0, 2, 1))
    return (new_points, re_flow)


# speedup vs baseline: 39.3084x; 1.0441x over previous
import functools

import jax
import jax.numpy as jnp
from jax import lax
from jax.experimental import pallas as pl
from jax.experimental.pallas import tpu as pltpu
from jax.experimental.pallas import tpu_sc as plsc

KNB = 9
_PREC = lax.Precision.HIGHEST


def _dot(a, b):
    return lax.dot_general(a.astype(jnp.bfloat16), b.astype(jnp.bfloat16),
                           (((1,), (0,)), ((), ())),
                           preferred_element_type=jnp.float32)


def _leaky(x):
    return jnp.where(x >= 0.0, x, 0.1 * x)


def _knn_body(n_total, rt, xr_ref, xam_ref, sq_ref, idx_ref):
    b = pl.program_id(0)
    xr = xr_ref[0]
    xam = xam_ref[0]
    sqm = sq_ref[0]
    sqr = jnp.sum(xr * xr, axis=0, keepdims=True)
    cross2 = _dot(xam * -2.0, xr)
    dist = (sqm + sqr) + cross2
    ng = n_total // 128
    cand = lax.broadcasted_iota(jnp.int32, (n_total, rt), 0)
    c7 = jnp.bitwise_and(cand, 127)
    bits = lax.bitcast_convert_type(jnp.maximum(dist, 1.0e-35), jnp.int32)
    keys = lax.bitcast_convert_type(
        jnp.bitwise_or(jnp.bitwise_and(bits, jnp.int32(-128)), c7),
        jnp.float32)
    g_iota = lax.broadcasted_iota(jnp.int32, (ng, rt), 0)
    gmin = jnp.min(keys.reshape(ng, 128, rt), axis=1)
    rows = []
    for j in range(KNB):
        m = jnp.min(gmin, axis=0, keepdims=True)
        gj = jnp.min(jnp.where(gmin == m, g_iota, ng), axis=0, keepdims=True)
        cj = jnp.bitwise_and(lax.bitcast_convert_type(m, jnp.int32), 127)
        sel = gj * 128 + cj
        rows.append(sel)
        if j + 1 < KNB:
            keys = jnp.where(cand == sel, 3.0e38, keys)
            gmin = jnp.min(keys.reshape(ng, 128, rt), axis=1)
    idx_ref[0] = jnp.concatenate(rows, axis=0) + b * n_total


def _knn(xyzC, xyzT, sqT):
    B, _, N = xyzC.shape
    RT = 512
    return pl.pallas_call(
        functools.partial(_knn_body, N, RT),
        grid=(B, N // RT),
        in_specs=[
            pl.BlockSpec((1, 3, RT), lambda b, i: (b, 0, i)),
            pl.BlockSpec((1, N, 3), lambda b, i: (b, 0, 0)),
            pl.BlockSpec((1, N, 1), lambda b, i: (b, 0, 0)),
        ],
        out_specs=pl.BlockSpec((1, KNB, RT), lambda b, i: (b, 0, i)),
        out_shape=jax.ShapeDtypeStruct((B, KNB, N), jnp.int32),
    )(xyzC, xyzT, sqT)


def _proj_body(xyz_ref, fe_ref, cv_ref, fl_ref, w1_ref, w2x_ref,
               q1_ref, px1_ref, px2_ref, sq_ref):
    xc = xyz_ref[0]
    w1 = w1_ref[...]
    px1 = _dot(w1[:, :3], xc)
    q1 = (px1 + _dot(w1[:, 3:131], fe_ref[0])
          + _dot(w1[:, 131:195], cv_ref[0])
          + _dot(w1[:, 195:], fl_ref[0]))
    px2 = _dot(w2x_ref[...], xc)
    q1_ref[0] = jnp.transpose(q1, (1, 0))
    px1_ref[0] = jnp.transpose(px1, (1, 0))
    px2_ref[0] = jnp.transpose(px2, (1, 0))
    sq_ref[0] = jnp.transpose(jnp.sum(xc * xc, axis=0, keepdims=True), (1, 0))


def _proj(xyz, feats, cost_volume, flow, W1, W2x):
    B, _, N = xyz.shape
    TN = 2048
    f32 = jnp.float32
    return pl.pallas_call(
        _proj_body,
        grid=(B, N // TN),
        in_specs=[
            pl.BlockSpec((1, 3, TN), lambda b, i: (b, 0, i)),
            pl.BlockSpec((1, 128, TN), lambda b, i: (b, 0, i)),
            pl.BlockSpec((1, 64, TN), lambda b, i: (b, 0, i)),
            pl.BlockSpec((1, 3, TN), lambda b, i: (b, 0, i)),
            pl.BlockSpec((128, 198), lambda b, i: (0, 0)),
            pl.BlockSpec((128, 3), lambda b, i: (0, 0)),
        ],
        out_specs=[pl.BlockSpec((1, TN, 128), lambda b, i: (b, i, 0))] * 3
        + [pl.BlockSpec((1, TN, 1), lambda b, i: (b, i, 0))],
        out_shape=[jax.ShapeDtypeStruct((B, N, 128), f32)] * 3
        + [jax.ShapeDtypeStruct((B, N, 1), f32)],
    )(xyz, feats, cost_volume, flow, W1, W2x)


_NC, _NS = 2, 16
_NW = _NC * _NS


def _gather(qT, idxf):
    R = idxf.shape[0]
    D = qT.shape[1]
    per_w = R // _NW
    CH = 512
    n_ch = per_w // CH
    mesh = plsc.VectorSubcoreMesh(core_axis_name="c", subcore_axis_name="s")

    def body(table_hbm, idx_hbm, out_hbm, idx_v, rows_v, sem):
        wid = lax.axis_index("s") * _NC + lax.axis_index("c")
        base = wid * per_w
        for it in range(n_ch):
            off = base + it * CH
            pltpu.sync_copy(idx_hbm.at[pl.ds(off, CH)], idx_v)
            pltpu.async_copy(table_hbm.at[idx_v], rows_v, sem).wait()
            pltpu.sync_copy(rows_v, out_hbm.at[pl.ds(off, CH)])

    f = pl.kernel(
        body,
        out_type=jax.ShapeDtypeStruct((R, D), jnp.float32),
        mesh=mesh,
        scratch_types=[
            pltpu.VMEM((CH,), jnp.int32),
            pltpu.VMEM((CH, D), jnp.float32),
            pltpu.SemaphoreType.DMA,
        ],
    )
    return f(qT, idxf)


def _reduce_body(g_ref, px_ref, b_ref, mx_ref, mn_ref, sp_ref, ssp_ref):
    g0 = g_ref[0]
    mx = g0
    mn = g0
    s = g0
    ss = g0 * g0
    for j in range(1, KNB):
        gj = g_ref[j]
        mx = jnp.maximum(mx, gj)
        mn = jnp.minimum(mn, gj)
        s = s + gj
        ss = ss + gj * gj
    mx_ref[...] = mx
    mn_ref[...] = mn
    pb = px_ref[...] - b_ref[...]
    kf = float(KNB)
    srow = s - kf * pb
    ssrow = ss - 2.0 * pb * s + kf * (pb * pb)
    sp_ref[...] = jnp.sum(srow, axis=0, keepdims=True)[None]
    ssp_ref[...] = jnp.sum(ssrow, axis=0, keepdims=True)[None]


def _kreduce(G3, pxT, bvec):
    _, M, D = G3.shape
    TP = 512
    NT = M // TP
    f32 = jnp.float32
    return pl.pallas_call(
        _reduce_body,
        grid=(NT,),
        in_specs=[
            pl.BlockSpec((KNB, TP, D), lambda i: (0, i, 0)),
            pl.BlockSpec((TP, D), lambda i: (i, 0)),
            pl.BlockSpec((1, D), lambda i: (0, 0)),
        ],
        out_specs=[
            pl.BlockSpec((TP, D), lambda i: (i, 0)),
            pl.BlockSpec((TP, D), lambda i: (i, 0)),
            pl.BlockSpec((1, 1, D), lambda i: (i, 0, 0)),
            pl.BlockSpec((1, 1, D), lambda i: (i, 0, 0)),
        ],
        out_shape=[
            jax.ShapeDtypeStruct((M, D), f32),
            jax.ShapeDtypeStruct((M, D), f32),
            jax.ShapeDtypeStruct((NT, 1, D), f32),
            jax.ShapeDtypeStruct((NT, 1, D), f32),
        ],
    )(G3, pxT, bvec)


def _bn_sel(count, sp, ssp, mx, mn, px, b, g, be):
    tot = jnp.sum(sp, axis=0)
    tot2 = jnp.sum(ssp, axis=0)
    mean = tot * (1.0 / count)
    var = tot2 * (1.0 / count) - mean * mean
    scale = g * lax.rsqrt(var + 1e-5)
    xsel = jnp.where(g >= 0.0, mx, mn) - (px - b)
    return _leaky((xsel - mean) * scale + be)


def _fin1_body(count, sp_ref, ssp_ref, mx_ref, mn_ref, px1_ref, b_ref, g_ref,
               be_ref, px2_ref, w2pT_ref, q2_ref):
    y = _bn_sel(count, sp_ref[...], ssp_ref[...], mx_ref[...], mn_ref[...],
                px1_ref[...], b_ref[...], g_ref[...], be_ref[...])
    q2_ref[...] = _dot(y, w2pT_ref[...]) + px2_ref[...]


def _fin1(count, sp, ssp, mx, mn, px1T, b1, g1, be1, px2T, w2pT):
    M, D = mx.shape
    TP = 2048
    NT1 = sp.shape[0]
    return pl.pallas_call(
        functools.partial(_fin1_body, count),
        grid=(M // TP,),
        in_specs=[
            pl.BlockSpec((NT1, 1, D), lambda i: (0, 0, 0)),
            pl.BlockSpec((NT1, 1, D), lambda i: (0, 0, 0)),
            pl.BlockSpec((TP, D), lambda i: (i, 0)),
            pl.BlockSpec((TP, D), lambda i: (i, 0)),
            pl.BlockSpec((TP, D), lambda i: (i, 0)),
            pl.BlockSpec((1, D), lambda i: (0, 0)),
            pl.BlockSpec((1, D), lambda i: (0, 0)),
            pl.BlockSpec((1, D), lambda i: (0, 0)),
            pl.BlockSpec((TP, D), lambda i: (i, 0)),
            pl.BlockSpec((D, D), lambda i: (0, 0)),
        ],
        out_specs=pl.BlockSpec((TP, D), lambda i: (i, 0)),
        out_shape=jax.ShapeDtypeStruct((M, D), jnp.float32),
    )(sp, ssp, mx, mn, px1T, b1, g1, be1, px2T, w2pT)


def _fin2_body(count, sp_ref, ssp_ref, mx_ref, mn_ref, px2_ref, b_ref, g_ref,
               be_ref, wm1T_ref, bm1_ref, wm2T_ref, bm2_ref, wfcT_ref,
               bfc_ref, np_ref, rf_ref):
    y = _bn_sel(count, sp_ref[...], ssp_ref[...], mx_ref[...], mn_ref[...],
                px2_ref[...], b_ref[...], g_ref[...], be_ref[...])
    h1 = _leaky(_dot(y, wm1T_ref[...]) + bm1_ref[...])
    h2 = _leaky(_dot(h1, wm2T_ref[...]) + bm2_ref[...])
    rf = _dot(h2, wfcT_ref[...]) + bfc_ref[...]
    np_ref[...] = h2
    rf_ref[...] = jnp.clip(rf, -20.0, 20.0)


def _fin2(count, sp, ssp, mx, mn, px2T, b2, g2, be2, wm1T, bm1, wm2T, bm2,
          wfcT, bfc):
    M, D = mx.shape
    TP = 2048
    NT1 = sp.shape[0]
    Dm = wm2T.shape[1]
    Do = wfcT.shape[1]
    return pl.pallas_call(
        functools.partial(_fin2_body, count),
        grid=(M // TP,),
        in_specs=[
            pl.BlockSpec((NT1, 1, D), lambda i: (0, 0, 0)),
            pl.BlockSpec((NT1, 1, D), lambda i: (0, 0, 0)),
            pl.BlockSpec((TP, D), lambda i: (i, 0)),
            pl.BlockSpec((TP, D), lambda i: (i, 0)),
            pl.BlockSpec((TP, D), lambda i: (i, 0)),
            pl.BlockSpec((1, D), lambda i: (0, 0)),
            pl.BlockSpec((1, D), lambda i: (0, 0)),
            pl.BlockSpec((1, D), lambda i: (0, 0)),
            pl.BlockSpec((D, D), lambda i: (0, 0)),
            pl.BlockSpec((1, D), lambda i: (0, 0)),
            pl.BlockSpec((D, Dm), lambda i: (0, 0)),
            pl.BlockSpec((1, Dm), lambda i: (0, 0)),
            pl.BlockSpec((Dm, Do), lambda i: (0, 0)),
            pl.BlockSpec((1, Do), lambda i: (0, 0)),
        ],
        out_specs=[
            pl.BlockSpec((TP, Dm), lambda i: (i, 0)),
            pl.BlockSpec((TP, Do), lambda i: (i, 0)),
        ],
        out_shape=[
            jax.ShapeDtypeStruct((M, Dm), jnp.float32),
            jax.ShapeDtypeStruct((M, Do), jnp.float32),
        ],
    )(sp, ssp, mx, mn, px2T, b2, g2, be2, wm1T, bm1, wm2T, bm2, wfcT, bfc)


def kernel(xyz, cost_volume, feats, flow, W1, b1, g1, be1, W2, b2, g2, be2,
           Wm1, bm1, Wm2, bm2, Wfc, bfc):
    B, _, N = xyz.shape
    M = B * N
    count = float(M * KNB)

    xyzT = jnp.transpose(xyz, (0, 2, 1))

    q1, px1, px2, sq = _proj(xyz, feats, cost_volume, flow, W1, W2[:, :3])
    q1T = q1.reshape(M, 128)
    px1T = px1.reshape(M, 128)
    px2T = px2.reshape(M, 128)

    idx = _knn(xyz, xyzT, sq)
    idxf = jnp.transpose(idx, (1, 0, 2)).reshape(-1)

    r1 = lambda v: v.reshape(1, -1)
    G1 = _gather(q1T, idxf).reshape(KNB, M, 128)
    mx1, mn1, sp1, ssp1 = _kreduce(G1, px1T, r1(b1))
    q2T = _fin1(count, sp1, ssp1, mx1, mn1, px1T, r1(b1), r1(g1), r1(be1),
                px2T, jnp.transpose(W2[:, 3:], (1, 0)))

    G2 = _gather(q2T, idxf).reshape(KNB, M, 128)
    mx2, mn2, sp2, ssp2 = _kreduce(G2, px2T, r1(b2))
    npT, rfT = _fin2(count, sp2, ssp2, mx2, mn2, px2T, r1(b2), r1(g2),
                     r1(be2), jnp.transpose(Wm1, (1, 0)), r1(bm1),
                     jnp.transpose(Wm2, (1, 0)), r1(bm2),
                     jnp.transpose(Wfc, (1, 0)), r1(bfc))

    new_points = jnp.transpose(npT.reshape(B, N, -1), (0, 2, 1))
    re_flow = jnp.transpose(rfT.reshape(B, N, -1), (0, 2, 1))
    return (new_points, re_flow)
